# Initial kernel scaffold; baseline (speedup 1.0000x reference)
import jax
import jax.numpy as jnp
from jax.experimental import pallas as pl


def kernel(pred_logits, pred_boxes, target_sizes):
    raise NotImplementedError("write your pallas kernel here")


# speedup vs baseline: 1.4389x; 1.4389x over previous
import functools

import jax
import jax.numpy as jnp
from jax.experimental import pallas as pl
from jax.experimental.pallas import tpu as pltpu

_NSEL = 300
_Q = 5000
_C = 91
_LANES = 128
_ROWS = 3584
_ROWS_HI = 28
_OUTPAD = 304


def _postprocess_kernel(prob_ref, boxes_ref, scale_ref,
                        scores_ref, labels_ref, boxout_ref,
                        xs_ref, rmax_ref, bxy_ref):
    xs_ref[...] = prob_ref[0]
    rmax_ref[...] = jnp.max(prob_ref[0].reshape(_ROWS_HI, _LANES, _LANES),
                            axis=2)

    bx = boxes_ref[0]
    i4 = jax.lax.broadcasted_iota(jnp.int32, (1, 4), 1)
    cx = bx[:, 0:1]
    cy = bx[:, 1:2]
    hw = 0.5 * bx[:, 2:3]
    hh = 0.5 * bx[:, 3:4]
    xyxy = jnp.where(i4 == 0, cx - hw,
                     jnp.where(i4 == 1, cy - hh,
                               jnp.where(i4 == 2, cx + hw, cy + hh)))
    bxy_ref[...] = xyxy * scale_ref[0]

    flat_iota = (jax.lax.broadcasted_iota(jnp.int32, (_ROWS_HI, _LANES), 0)
                 * _LANES
                 + jax.lax.broadcasted_iota(jnp.int32, (_ROWS_HI, _LANES), 1))
    lane_iota = jax.lax.broadcasted_iota(jnp.int32, (1, _LANES), 1)
    big = jnp.int32(2 ** 30)

    def body(j, carry):
        rm = rmax_ref[...]
        m = jnp.max(rm)
        r = jnp.min(jnp.where(rm == m, flat_iota, big))
        row = xs_ref[pl.ds(r, 1), :]
        c = jnp.min(jnp.where(row == m, lane_iota, big))
        flat = r * _LANES + c
        q = flat // _C
        lab = flat - q * _C

        scores_ref[0, pl.ds(j, 1), :] = jnp.reshape(m, (1, 1))
        labels_ref[0, pl.ds(j, 1), :] = jnp.reshape(lab, (1, 1))
        boxout_ref[0, pl.ds(j, 1), :] = bxy_ref[pl.ds(q, 1), :]

        newrow = jnp.where(lane_iota == c, jnp.float32(-1.0), row)
        xs_ref[pl.ds(r, 1), :] = newrow
        rmax_ref[...] = jnp.where(flat_iota == r, jnp.max(newrow), rm)
        return carry

    jax.lax.fori_loop(0, _NSEL, body, 0)


@jax.jit
def kernel(pred_logits, pred_boxes, target_sizes):
    B, Q, C = pred_logits.shape
    prob = jax.nn.sigmoid(pred_logits)
    flat = prob.reshape(B, Q * C)
    flat = jnp.pad(flat, ((0, 0), (0, _ROWS * _LANES - Q * C)),
                   constant_values=-1.0)
    prob3 = flat.reshape(B, _ROWS, _LANES)

    ts = target_sizes.astype(jnp.float32)
    scale = jnp.stack([ts[:, 1], ts[:, 0], ts[:, 1], ts[:, 0]], axis=1)
    scale = scale.reshape(B, 1, 4)

    scores, labels, boxes = pl.pallas_call(
        _postprocess_kernel,
        grid=(B,),
        in_specs=[
            pl.BlockSpec((1, _ROWS, _LANES), lambda b: (b, 0, 0)),
            pl.BlockSpec((1, Q, 4), lambda b: (b, 0, 0)),
            pl.BlockSpec((1, 1, 4), lambda b: (b, 0, 0)),
        ],
        out_specs=[
            pl.BlockSpec((1, _OUTPAD, 1), lambda b: (b, 0, 0)),
            pl.BlockSpec((1, _OUTPAD, 1), lambda b: (b, 0, 0)),
            pl.BlockSpec((1, _OUTPAD, 4), lambda b: (b, 0, 0)),
        ],
        out_shape=[
            jax.ShapeDtypeStruct((B, _OUTPAD, 1), jnp.float32),
            jax.ShapeDtypeStruct((B, _OUTPAD, 1), jnp.int32),
            jax.ShapeDtypeStruct((B, _OUTPAD, 4), jnp.float32),
        ],
        scratch_shapes=[
            pltpu.VMEM((_ROWS, _LANES), jnp.float32),
            pltpu.VMEM((_ROWS_HI, _LANES), jnp.float32),
            pltpu.VMEM((Q, 4), jnp.float32),
        ],
    )(prob3, pred_boxes, scale)

    return (scores[:, :_NSEL, 0], labels[:, :_NSEL, 0], boxes[:, :_NSEL, :])


# speedup vs baseline: 1.5785x; 1.0970x over previous
import functools

import jax
import jax.numpy as jnp
from jax.experimental import pallas as pl
from jax.experimental.pallas import tpu as pltpu

_NSEL = 300
_Q = 5000
_C = 91
_LANES = 128
_ROWS = 3584
_ROWS_HI = 28
_OUTPAD = 304


_IL = 2


def _postprocess_kernel(prob_ref, boxes_ref, scale_ref,
                        scores_ref, labels_ref, boxout_ref,
                        xs_ref, rmax_ref, bxy_ref):
    xs_ref[...] = prob_ref[...]
    rmax_ref[...] = jnp.max(
        prob_ref[...].reshape(_IL, _ROWS_HI, _LANES, _LANES), axis=3)

    bx = boxes_ref[...]
    i4 = jax.lax.broadcasted_iota(jnp.int32, (1, 1, 4), 2)
    cx = bx[:, :, 0:1]
    cy = bx[:, :, 1:2]
    hw = 0.5 * bx[:, :, 2:3]
    hh = 0.5 * bx[:, :, 3:4]
    xyxy = jnp.where(i4 == 0, cx - hw,
                     jnp.where(i4 == 1, cy - hh,
                               jnp.where(i4 == 2, cx + hw, cy + hh)))
    bxy_ref[...] = xyxy * scale_ref[...]

    flat_iota = (jax.lax.broadcasted_iota(jnp.int32, (_ROWS_HI, _LANES), 0)
                 * _LANES
                 + jax.lax.broadcasted_iota(jnp.int32, (_ROWS_HI, _LANES), 1))
    lane_iota = jax.lax.broadcasted_iota(jnp.int32, (1, _LANES), 1)
    big = jnp.int32(2 ** 30)

    def body(j, carry):
        for i in range(_IL):
            rm = rmax_ref[i]
            m = jnp.max(rm)
            r = jnp.min(jnp.where(rm == m, flat_iota, big))
            row = xs_ref[i, pl.ds(r, 1), :]
            c = jnp.min(jnp.where(row == m, lane_iota, big))
            flat = r * _LANES + c
            q = flat // _C
            lab = flat - q * _C

            scores_ref[i, pl.ds(j, 1), :] = jnp.reshape(m, (1, 1))
            labels_ref[i, pl.ds(j, 1), :] = jnp.reshape(lab, (1, 1))
            boxout_ref[i, pl.ds(j, 1), :] = bxy_ref[i, pl.ds(q, 1), :]

            newrow = jnp.where(lane_iota == c, jnp.float32(-1.0), row)
            xs_ref[i, pl.ds(r, 1), :] = newrow
            rmax_ref[i] = jnp.where(flat_iota == r, jnp.max(newrow), rm)
        return carry

    jax.lax.fori_loop(0, _NSEL, body, 0)


@jax.jit
def kernel(pred_logits, pred_boxes, target_sizes):
    B, Q, C = pred_logits.shape
    prob = jax.nn.sigmoid(pred_logits)
    flat = prob.reshape(B, Q * C)
    flat = jnp.pad(flat, ((0, 0), (0, _ROWS * _LANES - Q * C)),
                   constant_values=-1.0)
    prob3 = flat.reshape(B, _ROWS, _LANES)

    ts = target_sizes.astype(jnp.float32)
    scale = jnp.stack([ts[:, 1], ts[:, 0], ts[:, 1], ts[:, 0]], axis=1)
    scale = scale.reshape(B, 1, 4)

    scores, labels, boxes = pl.pallas_call(
        _postprocess_kernel,
        grid=(B // _IL,),
        in_specs=[
            pl.BlockSpec((_IL, _ROWS, _LANES), lambda b: (b, 0, 0)),
            pl.BlockSpec((_IL, Q, 4), lambda b: (b, 0, 0)),
            pl.BlockSpec((_IL, 1, 4), lambda b: (b, 0, 0)),
        ],
        out_specs=[
            pl.BlockSpec((_IL, _OUTPAD, 1), lambda b: (b, 0, 0)),
            pl.BlockSpec((_IL, _OUTPAD, 1), lambda b: (b, 0, 0)),
            pl.BlockSpec((_IL, _OUTPAD, 4), lambda b: (b, 0, 0)),
        ],
        out_shape=[
            jax.ShapeDtypeStruct((B, _OUTPAD, 1), jnp.float32),
            jax.ShapeDtypeStruct((B, _OUTPAD, 1), jnp.int32),
            jax.ShapeDtypeStruct((B, _OUTPAD, 4), jnp.float32),
        ],
        scratch_shapes=[
            pltpu.VMEM((_IL, _ROWS, _LANES), jnp.float32),
            pltpu.VMEM((_IL, _ROWS_HI, _LANES), jnp.float32),
            pltpu.VMEM((_IL, Q, 4), jnp.float32),
        ],
    )(prob3, pred_boxes, scale)

    return (scores[:, :_NSEL, 0], labels[:, :_NSEL, 0], boxes[:, :_NSEL, :])
